# Initial kernel scaffold; baseline (speedup 1.0000x reference)
import jax
import jax.numpy as jnp
from jax.experimental import pallas as pl


def kernel(anchors, deltas, objectness):
    raise NotImplementedError("write your pallas kernel here")


# speedup vs baseline: 58.2170x; 58.2170x over previous
import functools
import math

import jax
import jax.numpy as jnp
from jax import lax
from jax.experimental import pallas as pl

N_ANCHORS = 20000
B = 4
IMG = 800.0
PRE_NMS_TOP_N = 2000
POST_NMS_TOP_N = 2000
NMS_THRESH = 0.7
MIN_SIZE = 1.0
BBOX_XFORM_CLIP = math.log(1000.0 / 16.0)

NPAD = 20480
ROWS = NPAD // 128
TOP = 2048
CR = TOP // 128

_INTERPRET = False


def _iota2(shape):
    r = lax.broadcasted_iota(jnp.int32, shape, 0)
    c = lax.broadcasted_iota(jnp.int32, shape, 1)
    return r * shape[1] + c


def _cx_pass(arrs, j, want_first, cmp2):
    shape = arrs[0].shape
    i = _iota2(shape)
    bit0 = (i & j) == 0

    def partner(x):
        if j < 128:
            a = jnp.roll(x, -j, axis=1)
            b = jnp.roll(x, j, axis=1)
        else:
            jr = j // 128
            a = jnp.roll(x, -jr, axis=0)
            b = jnp.roll(x, jr, axis=0)
        return jnp.where(bit0, a, b)

    parts = [partner(x) for x in arrs]
    self_first = cmp2(arrs, parts)
    swap = jnp.logical_xor(want_first, self_first)
    return [jnp.where(swap, p, x) for x, p in zip(arrs, parts)]


def _cmp_topk(a, b):
    return (a[0] > b[0]) | ((a[0] == b[0]) & (a[1] < b[1]))


def _cmp_int(a, b):
    return a[0] < b[0]


def _bitonic_sort(arrs, cmp2):
    n = arrs[0].shape[0] * arrs[0].shape[1]
    i = _iota2(arrs[0].shape)
    k = 2
    while k <= n:
        j = k // 2
        while j >= 1:
            wf = ((i & j) == 0) == ((i & k) == 0)
            arrs = _cx_pass(arrs, j, wf, cmp2)
            j //= 2
        k *= 2
    return arrs


def _merge_top(a_arrs, b_desc_arrs, cmp2):
    n = a_arrs[0].shape[0] * a_arrs[0].shape[1]
    i = _iota2(a_arrs[0].shape)
    first = cmp2(a_arrs, b_desc_arrs)
    m = [jnp.where(first, x, y) for x, y in zip(a_arrs, b_desc_arrs)]
    j = n // 2
    while j >= 1:
        wf = (i & j) == 0
        m = _cx_pass(m, j, wf, cmp2)
        j //= 2
    return m


def _rpn_body(obj_ref, af_ref, df_ref, boxes_ref, sc_ref):
    f32 = jnp.float32

    local_iota = _iota2((CR, 128))

    def chunk_arrays(c):
        rows = pl.ds(c * CR, CR)
        key = obj_ref[0, rows, :]
        idx = c * TOP + local_iota
        pays = [af_ref[f, rows, :] for f in range(4)]
        pays += [df_ref[0, f, rows, :] for f in range(4)]
        return [key, idx] + pays

    carry = tuple(_bitonic_sort(chunk_arrays(0), _cmp_topk))

    def _cmp_topk_flip(a, b):
        return _cmp_topk(b, a)

    def mbody(c, carry):
        ch = _bitonic_sort(chunk_arrays(c), _cmp_topk_flip)
        return tuple(_merge_top(list(carry), ch, _cmp_topk))

    res = lax.fori_loop(1, NPAD // TOP, mbody, carry)
    score = res[0]
    ax1, ay1, ax2, ay2, dx, dy, dw, dh = res[2:]

    aw = ax2 - ax1
    ah = ay2 - ay1
    cx = ax1 + 0.5 * aw
    cy = ay1 + 0.5 * ah
    dwc = jnp.minimum(dw, BBOX_XFORM_CLIP)
    dhc = jnp.minimum(dh, BBOX_XFORM_CLIP)
    pcx = dx * aw + cx
    pcy = dy * ah + cy
    pw = jnp.exp(dwc) * aw
    ph = jnp.exp(dhc) * ah
    x1 = jnp.clip(pcx - 0.5 * pw, 0.0, IMG)
    y1 = jnp.clip(pcy - 0.5 * ph, 0.0, IMG)
    x2 = jnp.clip(pcx + 0.5 * pw, 0.0, IMG)
    y2 = jnp.clip(pcy + 0.5 * ph, 0.0, IMG)

    en = jnp.exp(-jnp.abs(score))
    prob = jnp.where(score >= 0.0, 1.0 / (1.0 + en), en / (1.0 + en))

    pos = local_iota
    ws = x2 - x1
    hs = y2 - y1
    valid = (ws >= MIN_SIZE) & (hs >= MIN_SIZE) & (prob > 0.0) & (pos < PRE_NMS_TOP_N)
    validf = valid.astype(f32)
    area = ws * hs

    r128 = lax.broadcasted_iota(jnp.int32, (128, 128), 0)
    c128 = lax.broadcasted_iota(jnp.int32, (128, 128), 1)
    ident = (r128 == c128).astype(f32)
    tri = (r128 < c128).astype(f32)

    def tcol(v):
        return lax.dot_general(ident, v, (((1,), (1,)), ((), ())),
                               preferred_element_type=f32,
                               precision=lax.Precision.HIGHEST)

    x1t, y1t, x2t, y2t = tcol(x1), tcol(y1), tcol(x2), tcol(y2)
    areat = tcol(area)
    validt = tcol(validf)

    def iou_mat(J, I):
        ltx = jnp.maximum(x1t[:, J:J + 1], x1[I:I + 1, :])
        lty = jnp.maximum(y1t[:, J:J + 1], y1[I:I + 1, :])
        rbx = jnp.minimum(x2t[:, J:J + 1], x2[I:I + 1, :])
        rby = jnp.minimum(y2t[:, J:J + 1], y2[I:I + 1, :])
        iw = jnp.maximum(rbx - ltx, 0.0)
        ih = jnp.maximum(rby - lty, 0.0)
        inter = iw * ih
        denom = areat[:, J:J + 1] + area[I:I + 1, :] - inter + 1e-9
        return inter / denom

    keepcols = []
    keeprows = []
    for I in range(CR):
        supf = jnp.zeros((1, 128), f32)
        for J in range(I):
            hit = (iou_mat(J, I) > NMS_THRESH).astype(f32) * keepcols[J]
            supf = jnp.maximum(supf, jnp.max(hit, axis=0, keepdims=True))
        ntl = ((iou_mat(I, I) > NMS_THRESH).astype(f32)
               * tri * validt[:, I:I + 1])
        init = validf[I:I + 1, :] * (1.0 - supf)

        def fcond(c):
            return c[1]

        def fbody(c, ntl=ntl, init=init):
            kp, _ = c
            conf = jnp.max(ntl * tcol(kp), axis=0, keepdims=True)
            new = init * (1.0 - conf)
            return new, jnp.any(new != kp)

        keep_i, _ = lax.while_loop(fcond, fbody,
                                   (init, jnp.asarray(True)))
        keeprows.append(keep_i)
        keepcols.append(tcol(keep_i))

    keep = jnp.concatenate(keeprows, axis=0)
    final = keep > 0.0

    outsc = jnp.where(final, prob, -1.0)
    karr = pos + jnp.where(final, 0, jnp.where(valid, TOP, 3 * TOP))
    part = _bitonic_sort([karr, x1, y1, x2, y2, outsc], _cmp_int)
    boxes_ref[0, 0] = part[1]
    boxes_ref[0, 1] = part[2]
    boxes_ref[0, 2] = part[3]
    boxes_ref[0, 3] = part[4]
    sc_ref[0] = part[5]


@jax.jit
def kernel(anchors, deltas, objectness):
    b = objectness.shape[0]
    obj_p = jnp.pad(objectness, ((0, 0), (0, NPAD - N_ANCHORS)),
                    constant_values=-jnp.inf).reshape(b, ROWS, 128)
    af = jnp.pad(anchors, ((0, NPAD - N_ANCHORS), (0, 0))).T.reshape(4, ROWS, 128)
    df = jnp.pad(deltas, ((0, 0), (0, NPAD - N_ANCHORS), (0, 0))
                 ).transpose(0, 2, 1).reshape(b, 4, ROWS, 128)

    boxes_f, scores_f = pl.pallas_call(
        _rpn_body,
        grid=(b,),
        in_specs=[
            pl.BlockSpec((1, ROWS, 128), lambda i: (i, 0, 0)),
            pl.BlockSpec((4, ROWS, 128), lambda i: (0, 0, 0)),
            pl.BlockSpec((1, 4, ROWS, 128), lambda i: (i, 0, 0, 0)),
        ],
        out_specs=[
            pl.BlockSpec((1, 4, CR, 128), lambda i: (i, 0, 0, 0)),
            pl.BlockSpec((1, CR, 128), lambda i: (i, 0, 0)),
        ],
        out_shape=[
            jax.ShapeDtypeStruct((b, 4, CR, 128), jnp.float32),
            jax.ShapeDtypeStruct((b, CR, 128), jnp.float32),
        ],
        interpret=_INTERPRET,
    )(obj_p, af, df)

    boxes = boxes_f.reshape(b, 4, TOP)[:, :, :POST_NMS_TOP_N].transpose(0, 2, 1)
    scores = scores_f.reshape(b, TOP)[:, :POST_NMS_TOP_N]
    return boxes, scores
